# baseline (speedup 1.0000x reference)
import functools
import math

import jax
import jax.numpy as jnp
from jax import lax
from jax.experimental import pallas as pl
from jax.experimental.pallas import tpu as pltpu
from jax.experimental.pallas import tpu_sc as plsc

VOCAB = 1000000
HIDDEN = 64
SCALE = math.sqrt(HIDDEN)

ROWS = 4096
COLS = 200

NUM_CORES = 2
NUM_SUBCORES = 16
NW = NUM_CORES * NUM_SUBCORES
RPW = ROWS // NW

_mesh = plsc.VectorSubcoreMesh(core_axis_name="c", subcore_axis_name="s")


@functools.partial(
    pl.kernel,
    mesh=_mesh,
    out_type=jax.ShapeDtypeStruct((ROWS, COLS, HIDDEN), jnp.float32),
    scratch_types=[
        pltpu.VMEM((RPW, COLS), jnp.int32),
        pltpu.VMEM((COLS, HIDDEN), jnp.float32),
        pltpu.VMEM((COLS, HIDDEN), jnp.float32),
        pltpu.VMEM((COLS, HIDDEN), jnp.float32),
        pltpu.VMEM((COLS, HIDDEN), jnp.float32),
        pltpu.SemaphoreType.DMA,
        pltpu.SemaphoreType.DMA,
        pltpu.SemaphoreType.DMA,
        pltpu.SemaphoreType.DMA,
    ],
    compiler_params=pltpu.CompilerParams(use_tc_tiling_on_sc=False),
)
def _emb_lookup(table_hbm, idx_hbm, out_hbm, idx_v, g0, g1, w0, w1,
                gsem0, gsem1, wsem0, wsem1):
    wid = lax.axis_index("s") * NUM_CORES + lax.axis_index("c")
    base = wid * RPW

    def scale(src, dst):
        @plsc.parallel_loop(0, COLS, unroll=4)
        def _(r):
            for j in range(HIDDEN // 16):
                sl = pl.ds(j * 16, 16)
                dst[r, sl] = src[r, sl] * SCALE

    def step(i, gb, wb, gsem, wsem, wait_wb, issue_next):
        pltpu.make_async_copy(table_hbm.at[idx_v.at[i]], gb, gsem).wait()
        if wait_wb:
            pltpu.make_async_copy(wb, out_hbm.at[base + i - 2], wsem).wait()
        scale(gb, wb)
        if issue_next:
            pltpu.async_copy(table_hbm.at[idx_v.at[i + 2]], gb, gsem)
        pltpu.async_copy(wb, out_hbm.at[base + i], wsem)

    pltpu.sync_copy(idx_hbm.at[pl.ds(base, RPW)], idx_v)

    pltpu.async_copy(table_hbm.at[idx_v.at[0]], g0, gsem0)
    pltpu.async_copy(table_hbm.at[idx_v.at[1]], g1, gsem1)

    step(0, g0, w0, gsem0, wsem0, wait_wb=False, issue_next=True)
    step(1, g1, w1, gsem1, wsem1, wait_wb=False, issue_next=True)

    def group_body(g, carry):
        i = g * 2
        step(i, g0, w0, gsem0, wsem0, wait_wb=True, issue_next=True)
        step(i + 1, g1, w1, gsem1, wsem1, wait_wb=True, issue_next=True)
        return carry

    lax.fori_loop(1, RPW // 2 - 1, group_body, 0)

    step(RPW - 2, g0, w0, gsem0, wsem0, wait_wb=True, issue_next=False)
    step(RPW - 1, g1, w1, gsem1, wsem1, wait_wb=True, issue_next=False)

    pltpu.make_async_copy(w0, out_hbm.at[base + RPW - 2], wsem0).wait()
    pltpu.make_async_copy(w1, out_hbm.at[base + RPW - 1], wsem1).wait()


def kernel(x, emb_weight):
    return _emb_lookup(emb_weight, x.astype(jnp.int32))


# speedup vs baseline: 1.0195x; 1.0195x over previous
import functools
import math

import jax
import jax.numpy as jnp
from jax import lax
from jax.experimental import pallas as pl
from jax.experimental.pallas import tpu as pltpu
from jax.experimental.pallas import tpu_sc as plsc

VOCAB = 1000000
HIDDEN = 64
SCALE = math.sqrt(HIDDEN)

ROWS = 4096
COLS = 200
B = ROWS * COLS

NUM_CORES = 2
NUM_SUBCORES = 16
NW = NUM_CORES * NUM_SUBCORES
RPW = ROWS // NW
IPW = RPW * COLS

_mesh = plsc.VectorSubcoreMesh(core_axis_name="c", subcore_axis_name="s")


@functools.partial(
    pl.kernel,
    mesh=_mesh,
    out_type=jax.ShapeDtypeStruct((ROWS, COLS, HIDDEN), jnp.float32),
    scratch_types=[
        pltpu.VMEM((IPW,), jnp.int32),
        pltpu.VMEM((COLS,), jnp.int32),
        pltpu.VMEM((COLS,), jnp.int32),
        pltpu.VMEM((COLS, 2 * HIDDEN), jnp.float32),
        pltpu.VMEM((COLS, 2 * HIDDEN), jnp.float32),
        pltpu.VMEM((COLS, HIDDEN), jnp.float32),
        pltpu.VMEM((COLS, HIDDEN), jnp.float32),
        pltpu.SemaphoreType.DMA,
        pltpu.SemaphoreType.DMA,
        pltpu.SemaphoreType.DMA,
        pltpu.SemaphoreType.DMA,
    ],
    compiler_params=pltpu.CompilerParams(use_tc_tiling_on_sc=True),
)
def _emb_lookup(table_hbm, idx_hbm, out_hbm, raw_v, h0, h1, g0, g1, w0, w1,
                gsem0, gsem1, wsem0, wsem1):
    wid = lax.axis_index("s") * NUM_CORES + lax.axis_index("c")
    base = wid * RPW

    def halve(i, hb):
        for o in list(range(0, COLS - 16, 16)) + [COLS - 16]:
            v = raw_v[pl.ds(i * COLS + o, 16)]
            hb[pl.ds(o, 16)] = lax.shift_right_logical(v, 1)

    def scale(i, src, dst):
        def group16(o):
            par = (raw_v[pl.ds(i * COLS + o, 16)] & 1) * HIDDEN
            for k in range(16):
                off = par[k]
                for j in range(HIDDEN // 16):
                    dst[o + k, pl.ds(j * 16, 16)] = (
                        src[o + k, pl.ds(off + j * 16, 16)] * SCALE
                    )

        @plsc.parallel_loop(0, COLS // 16, unroll=1)
        def _(gi):
            group16(gi * 16)

        group16(COLS - 16)

    def step(i, hb, gb, wb, gsem, wsem, wait_wb, issue_next):
        pltpu.make_async_copy(table_hbm.at[hb], gb, gsem).wait()
        if wait_wb:
            pltpu.make_async_copy(wb, out_hbm.at[base + i - 2], wsem).wait()
        scale(i, gb, wb)
        if issue_next:
            halve(i + 2, hb)
            pltpu.async_copy(table_hbm.at[hb], gb, gsem)
        pltpu.async_copy(wb, out_hbm.at[base + i], wsem)

    pltpu.sync_copy(idx_hbm.at[pl.ds(wid * IPW, IPW)], raw_v)

    halve(0, h0)
    pltpu.async_copy(table_hbm.at[h0], g0, gsem0)
    halve(1, h1)
    pltpu.async_copy(table_hbm.at[h1], g1, gsem1)

    step(0, h0, g0, w0, gsem0, wsem0, wait_wb=False, issue_next=True)
    step(1, h1, g1, w1, gsem1, wsem1, wait_wb=False, issue_next=True)

    def group_body(g, carry):
        i = g * 2
        step(i, h0, g0, w0, gsem0, wsem0, wait_wb=True, issue_next=True)
        step(i + 1, h1, g1, w1, gsem1, wsem1, wait_wb=True, issue_next=True)
        return carry

    lax.fori_loop(1, RPW // 2 - 1, group_body, 0)

    step(RPW - 2, h0, g0, w0, gsem0, wsem0, wait_wb=True, issue_next=False)
    step(RPW - 1, h1, g1, w1, gsem1, wsem1, wait_wb=True, issue_next=False)

    pltpu.make_async_copy(w0, out_hbm.at[base + RPW - 2], wsem0).wait()
    pltpu.make_async_copy(w1, out_hbm.at[base + RPW - 1], wsem1).wait()


def kernel(x, emb_weight):
    table2 = emb_weight.reshape(VOCAB // 2, 2 * HIDDEN)
    idx = x.reshape(B).astype(jnp.int32)
    return _emb_lookup(table2, idx)
